# Initial kernel scaffold; baseline (speedup 1.0000x reference)
import jax
import jax.numpy as jnp
from jax.experimental import pallas as pl


def kernel(x, edge_index, edge_weight, eps, W1, W2, W3):
    raise NotImplementedError("write your pallas kernel here")


# speedup vs baseline: 7.2935x; 7.2935x over previous
import functools

import jax
import jax.numpy as jnp
from jax import lax
from jax.experimental import pallas as pl
from jax.experimental.pallas import tpu as pltpu
from jax.experimental.pallas import tpu_sc as plsc

N = 10000
E = 320000
D_IN = 128
H1 = 32
H2 = 16

NC = 2
NS = 16
NW = NC * NS
EPW = E // NW
CH = 80
NCH = EPW // CH
NPAD = 10240
RPT = NPAD // NS


def _spmm_sc(h, src_r, dst_r, w_r):
  mesh = plsc.VectorSubcoreMesh(core_axis_name="c", subcore_axis_name="s")

  @functools.partial(
      pl.kernel,
      out_type=jax.ShapeDtypeStruct((NC, NPAD, H1), jnp.float32),
      mesh=mesh,
      compiler_params=pltpu.CompilerParams(
          use_tc_tiling_on_sc=False, needs_layout_passes=False),
      scratch_types=[
          pltpu.VMEM((NCH, CH), jnp.int32),
          pltpu.VMEM((NCH, CH), jnp.int32),
          pltpu.VMEM((NCH, CH), jnp.float32),
          pltpu.VMEM((CH, H1), jnp.float32),
          pltpu.VMEM((RPT, H1), jnp.float32),
          pltpu.VMEM_SHARED((NPAD, H1), jnp.float32),
          pltpu.SemaphoreType.DMA,
      ],
  )
  def k(h_hbm, src_hbm, dst_hbm, w_hbm, out_hbm,
        src_v, dst_v, w_v, rows_v, stage_v, acc_sh, sem):
    cid = lax.axis_index("c")
    sid = lax.axis_index("s")
    wid = cid * NS + sid

    zero16 = jnp.zeros((16,), jnp.float32)

    def zbody(i, carry):
      stage_v[i, pl.ds(0, 16)] = zero16
      stage_v[i, pl.ds(16, 16)] = zero16
      return carry

    lax.fori_loop(0, RPT, zbody, 0)
    pltpu.sync_copy(stage_v, acc_sh.at[pl.ds(sid * RPT, RPT)])

    pltpu.sync_copy(src_hbm.at[wid], src_v)
    pltpu.sync_copy(dst_hbm.at[wid], dst_v)
    pltpu.sync_copy(w_hbm.at[wid], w_v)
    plsc.subcore_barrier()

    lanes = lax.iota(jnp.int32, 16)

    def cbody(c, carry):
      pltpu.async_copy(h_hbm.at[src_v.at[c]], rows_v, sem).wait()

      def sbody(b, carry2):
        w16 = w_v[c, pl.ds(b * 16, 16)]
        for l in range(16):
          s = jnp.sum(jnp.where(lanes == l, w16, 0.0))
          wv = lax.broadcast(s, (16,))
          e = b * 16 + l
          rows_v[e, pl.ds(0, 16)] = rows_v[e, pl.ds(0, 16)] * wv
          rows_v[e, pl.ds(16, 16)] = rows_v[e, pl.ds(16, 16)] * wv
        return carry2

      lax.fori_loop(0, CH // 16, sbody, 0)
      pltpu.sync_copy(rows_v, acc_sh.at[dst_v.at[c]], add=True)
      return carry

    lax.fori_loop(0, NCH, cbody, 0)
    plsc.subcore_barrier()

    pltpu.sync_copy(acc_sh.at[pl.ds(sid * RPT, RPT)], stage_v)
    pltpu.sync_copy(stage_v, out_hbm.at[cid, pl.ds(sid * RPT, RPT)])

  return k(h, src_r, dst_r, w_r)


def _mm1_tc(x, W1):
  def body(x_ref, w_ref, o_ref):
    o_ref[...] = jnp.dot(x_ref[...], w_ref[...],
                         preferred_element_type=jnp.float32)

  return pl.pallas_call(
      body,
      grid=(10,),
      in_specs=[
          pl.BlockSpec((N // 10, D_IN), lambda i: (i, 0)),
          pl.BlockSpec((D_IN, H1), lambda i: (0, 0)),
      ],
      out_specs=pl.BlockSpec((N // 10, H1), lambda i: (i, 0)),
      out_shape=jax.ShapeDtypeStruct((N, H1), jnp.float32),
  )(x, W1)


def _relu_combine_tc(p):
  def body(p_ref, o_ref):
    o_ref[...] = jnp.maximum(p_ref[0] + p_ref[1], 0.0)

  blk = NPAD // 10
  return pl.pallas_call(
      body,
      grid=(10,),
      in_specs=[pl.BlockSpec((NC, blk, H1), lambda i: (0, i, 0))],
      out_specs=pl.BlockSpec((blk, H1), lambda i: (i, 0)),
      out_shape=jax.ShapeDtypeStruct((NPAD, H1), jnp.float32),
  )(p)


def _finalize_tc(q, eps, W2, W3):
  def body(q_ref, eps_ref, w2_ref, w3_ref, mu_ref, lv_ref, z_ref):
    s = q_ref[0] + q_ref[1]
    mu = jnp.dot(s, w2_ref[...], preferred_element_type=jnp.float32)
    lv = jnp.dot(s, w3_ref[...], preferred_element_type=jnp.float32)
    mu_ref[...] = mu
    lv_ref[...] = lv
    z_ref[...] = eps_ref[...] * jnp.exp(lv) + mu

  blk = NPAD // 10
  return pl.pallas_call(
      body,
      grid=(10,),
      in_specs=[
          pl.BlockSpec((NC, blk, H1), lambda i: (0, i, 0)),
          pl.BlockSpec((blk, H2), lambda i: (i, 0)),
          pl.BlockSpec((H1, H2), lambda i: (0, 0)),
          pl.BlockSpec((H1, H2), lambda i: (0, 0)),
      ],
      out_specs=[
          pl.BlockSpec((blk, H2), lambda i: (i, 0)),
          pl.BlockSpec((blk, H2), lambda i: (i, 0)),
          pl.BlockSpec((blk, H2), lambda i: (i, 0)),
      ],
      out_shape=[
          jax.ShapeDtypeStruct((N, H2), jnp.float32),
          jax.ShapeDtypeStruct((N, H2), jnp.float32),
          jax.ShapeDtypeStruct((N, H2), jnp.float32),
      ],
  )(q, eps, W2, W3)


def _gram_tc(z):
  BR = 1024
  BCOL = 1024
  gr = pl.cdiv(N, BR)
  gc = pl.cdiv(N, BCOL)

  def body(zr_ref, zc_ref, o_ref):
    o_ref[...] = lax.dot_general(
        zr_ref[...], zc_ref[...],
        (((1,), (1,)), ((), ())),
        preferred_element_type=jnp.float32)

  return pl.pallas_call(
      body,
      grid=(gr, gc),
      in_specs=[
          pl.BlockSpec((BR, H2), lambda i, j: (i, 0)),
          pl.BlockSpec((BCOL, H2), lambda i, j: (j, 0)),
      ],
      out_specs=pl.BlockSpec((BR, BCOL), lambda i, j: (i, j)),
      out_shape=jax.ShapeDtypeStruct((N, N), jnp.float32),
  )(z, z)


def kernel(x, edge_index, edge_weight, eps, W1, W2, W3):
  src = edge_index[0].astype(jnp.int32).reshape(NW, NCH, CH)
  dst = edge_index[1].astype(jnp.int32).reshape(NW, NCH, CH)
  w = edge_weight.reshape(NW, NCH, CH)

  t1 = _mm1_tc(x, W1)
  p = _spmm_sc(t1, src, dst, w)
  h1 = _relu_combine_tc(p)
  q = _spmm_sc(h1, src, dst, w)
  mu, logvar, z = _finalize_tc(q, eps, W2, W3)
  adj = _gram_tc(z)
  return (adj, mu, logvar)


# speedup vs baseline: 7.4814x; 1.0258x over previous
import functools

import jax
import jax.numpy as jnp
from jax import lax
from jax.experimental import pallas as pl
from jax.experimental.pallas import tpu as pltpu
from jax.experimental.pallas import tpu_sc as plsc

N = 10000
E = 320000
D_IN = 128
H1 = 32
H2 = 16

NC = 2
NS = 16
NW = NC * NS
EPW = E // NW
CH = 80
NCH = EPW // CH
NPAD = 10240
RPT = NPAD // NS


def _spmm_sc(h, src_r, dst_r, wb_r):
  mesh = plsc.VectorSubcoreMesh(core_axis_name="c", subcore_axis_name="s")

  @functools.partial(
      pl.kernel,
      out_type=jax.ShapeDtypeStruct((NC, NPAD, H1), jnp.float32),
      mesh=mesh,
      compiler_params=pltpu.CompilerParams(
          use_tc_tiling_on_sc=False, needs_layout_passes=False),
      scratch_types=[
          pltpu.VMEM((NCH, CH), jnp.int32),
          pltpu.VMEM((NCH, CH), jnp.int32),
          pltpu.VMEM((CH, H1), jnp.float32),
          pltpu.VMEM((CH, H1), jnp.float32),
          pltpu.VMEM((CH, 16), jnp.float32),
          pltpu.VMEM((CH, 16), jnp.float32),
          pltpu.VMEM((CH, H1), jnp.float32),
          pltpu.VMEM((RPT, H1), jnp.float32),
          pltpu.VMEM_SHARED((NPAD, H1), jnp.float32),
          pltpu.SemaphoreType.DMA,
          pltpu.SemaphoreType.DMA,
      ],
  )
  def k(h_hbm, src_hbm, dst_hbm, wb_hbm, out_hbm,
        src_v, dst_v, rows0, rows1, wb0, wb1, wrows, stage_v, acc_sh,
        sem0, sem1):
    cid = lax.axis_index("c")
    sid = lax.axis_index("s")
    wid = cid * NS + sid
    rows_b = (rows0, rows1)
    wb_b = (wb0, wb1)
    sems = (sem0, sem1)

    zero16 = jnp.zeros((16,), jnp.float32)

    def zbody(i, carry):
      stage_v[i, pl.ds(0, 16)] = zero16
      stage_v[i, pl.ds(16, 16)] = zero16
      return carry

    lax.fori_loop(0, RPT, zbody, 0)
    pltpu.sync_copy(stage_v, acc_sh.at[pl.ds(sid * RPT, RPT)])

    pltpu.sync_copy(src_hbm.at[wid], src_v)
    pltpu.sync_copy(dst_hbm.at[wid], dst_v)
    plsc.subcore_barrier()

    def start_gather(c, par):
      pltpu.async_copy(h_hbm.at[src_v.at[c]], rows_b[par], sems[par])
      pltpu.async_copy(wb_hbm.at[wid, c], wb_b[par], sems[par])

    def wait_gather(c, par):
      pltpu.make_async_copy(
          h_hbm.at[src_v.at[c]], rows_b[par], sems[par]).wait()
      pltpu.make_async_copy(
          wb_hbm.at[wid, c], wb_b[par], sems[par]).wait()

    def process(par):
      def sbody(e, carry):
        wv = wb_b[par][e, pl.ds(0, 16)]
        wrows[e, pl.ds(0, 16)] = rows_b[par][e, pl.ds(0, 16)] * wv
        wrows[e, pl.ds(16, 16)] = rows_b[par][e, pl.ds(16, 16)] * wv
        return carry

      lax.fori_loop(0, CH, sbody, 0, unroll=8)

    start_gather(0, 0)
    start_gather(1, 1)

    def kbody(kk, carry):
      for par in range(2):
        c = 2 * kk + par

        @pl.when(c < NCH)
        def _():
          wait_gather(c, par)
          process(par)

          @pl.when(c + 2 < NCH)
          def _():
            start_gather(c + 2, par)

          pltpu.sync_copy(wrows, acc_sh.at[dst_v.at[c]], add=True)

      return carry

    lax.fori_loop(0, (NCH + 1) // 2, kbody, 0)
    plsc.subcore_barrier()

    pltpu.sync_copy(acc_sh.at[pl.ds(sid * RPT, RPT)], stage_v)
    pltpu.sync_copy(stage_v, out_hbm.at[cid, pl.ds(sid * RPT, RPT)])

  return k(h, src_r, dst_r, wb_r)


def _mm1_tc(x, W1):
  def body(x_ref, w_ref, o_ref):
    o_ref[...] = jnp.dot(x_ref[...], w_ref[...],
                         preferred_element_type=jnp.float32)

  return pl.pallas_call(
      body,
      grid=(10,),
      in_specs=[
          pl.BlockSpec((N // 10, D_IN), lambda i: (i, 0)),
          pl.BlockSpec((D_IN, H1), lambda i: (0, 0)),
      ],
      out_specs=pl.BlockSpec((N // 10, H1), lambda i: (i, 0)),
      out_shape=jax.ShapeDtypeStruct((N, H1), jnp.float32),
  )(x, W1)


def _relu_combine_tc(p):
  def body(p_ref, o_ref):
    o_ref[...] = jnp.maximum(p_ref[0] + p_ref[1], 0.0)

  blk = NPAD // 10
  return pl.pallas_call(
      body,
      grid=(10,),
      in_specs=[pl.BlockSpec((NC, blk, H1), lambda i: (0, i, 0))],
      out_specs=pl.BlockSpec((blk, H1), lambda i: (i, 0)),
      out_shape=jax.ShapeDtypeStruct((NPAD, H1), jnp.float32),
  )(p)


def _finalize_tc(q, eps, W2, W3):
  def body(q_ref, eps_ref, w2_ref, w3_ref, mu_ref, lv_ref, z_ref):
    s = q_ref[0] + q_ref[1]
    mu = jnp.dot(s, w2_ref[...], preferred_element_type=jnp.float32)
    lv = jnp.dot(s, w3_ref[...], preferred_element_type=jnp.float32)
    mu_ref[...] = mu
    lv_ref[...] = lv
    z_ref[...] = eps_ref[...] * jnp.exp(lv) + mu

  blk = NPAD // 10
  return pl.pallas_call(
      body,
      grid=(10,),
      in_specs=[
          pl.BlockSpec((NC, blk, H1), lambda i: (0, i, 0)),
          pl.BlockSpec((blk, H2), lambda i: (i, 0)),
          pl.BlockSpec((H1, H2), lambda i: (0, 0)),
          pl.BlockSpec((H1, H2), lambda i: (0, 0)),
      ],
      out_specs=[
          pl.BlockSpec((blk, H2), lambda i: (i, 0)),
          pl.BlockSpec((blk, H2), lambda i: (i, 0)),
          pl.BlockSpec((blk, H2), lambda i: (i, 0)),
      ],
      out_shape=[
          jax.ShapeDtypeStruct((N, H2), jnp.float32),
          jax.ShapeDtypeStruct((N, H2), jnp.float32),
          jax.ShapeDtypeStruct((N, H2), jnp.float32),
      ],
  )(q, eps, W2, W3)


def _gram_tc(z):
  BR = 1024
  BCOL = 1024
  gr = pl.cdiv(N, BR)
  gc = pl.cdiv(N, BCOL)

  def body(zr_ref, zc_ref, o_ref):
    o_ref[...] = lax.dot_general(
        zr_ref[...], zc_ref[...],
        (((1,), (1,)), ((), ())),
        preferred_element_type=jnp.float32)

  return pl.pallas_call(
      body,
      grid=(gr, gc),
      in_specs=[
          pl.BlockSpec((BR, H2), lambda i, j: (i, 0)),
          pl.BlockSpec((BCOL, H2), lambda i, j: (j, 0)),
      ],
      out_specs=pl.BlockSpec((BR, BCOL), lambda i, j: (i, j)),
      out_shape=jax.ShapeDtypeStruct((N, N), jnp.float32),
  )(z, z)


def kernel(x, edge_index, edge_weight, eps, W1, W2, W3):
  src = edge_index[0].astype(jnp.int32).reshape(NW, NCH, CH)
  dst = edge_index[1].astype(jnp.int32).reshape(NW, NCH, CH)
  wb = jnp.broadcast_to(
      edge_weight.reshape(NW, NCH, CH, 1), (NW, NCH, CH, 16))

  t1 = _mm1_tc(x, W1)
  p = _spmm_sc(t1, src, dst, wb)
  h1 = _relu_combine_tc(p)
  q = _spmm_sc(h1, src, dst, wb)
  mu, logvar, z = _finalize_tc(q, eps, W2, W3)
  adj = _gram_tc(z)
  return (adj, mu, logvar)
